# baseline (speedup 1.0000x reference)
import functools

import jax
import jax.numpy as jnp
from jax import lax
from jax.experimental import pallas as pl
from jax.experimental.pallas import tpu as pltpu
from jax.experimental.pallas import tpu_sc as plsc

_NC = 2
_NS = 16
_NW = _NC * _NS
_D = 128
_NV = _D // 16
_EPS = 1e-12


def _make_fused(n_rows: int, T: int):
    rows_per_w = n_rows // _NW
    n_chunks = rows_per_w // T
    mesh = plsc.VectorSubcoreMesh(core_axis_name="c", subcore_axis_name="s")

    @functools.partial(
        pl.kernel,
        out_type=jax.ShapeDtypeStruct((n_rows, _D), jnp.float32),
        mesh=mesh,
        scratch_types=[
            pltpu.VMEM((rows_per_w,), jnp.int32),
            pltpu.VMEM((T, _D), jnp.float32),
            pltpu.VMEM((T, _D), jnp.float32),
            pltpu.VMEM((2, _D), jnp.float32),
            pltpu.VMEM((_D,), jnp.float32),
            pltpu.VMEM((_D,), jnp.float32),
            pltpu.SemaphoreType.DMA,
        ],
    )
    def fused_kernel(ids_hbm, table_hbm, pos_hbm, tt_hbm, gamma_hbm, beta_hbm,
                     out_hbm, idx_v, buf, bias_v, tt_v, gamma_v, beta_v, sem):
        wid = lax.axis_index("s") * _NC + lax.axis_index("c")
        base = wid * rows_per_w
        pltpu.sync_copy(ids_hbm.at[pl.ds(base, rows_per_w)], idx_v)
        pltpu.sync_copy(pos_hbm.at[pl.ds(0, T)], bias_v)
        pltpu.sync_copy(tt_hbm, tt_v)
        pltpu.sync_copy(gamma_hbm, gamma_v)
        pltpu.sync_copy(beta_hbm, beta_v)

        tt_row = [tt_v[0, pl.ds(16 * j, 16)] for j in range(_NV)]

        def bias_body(r, c):
            for j in range(_NV):
                sl = pl.ds(16 * j, 16)
                bias_v[r, sl] = bias_v[r, sl] + tt_row[j]
            return c

        lax.fori_loop(0, T, bias_body, 0)

        g_vec = [gamma_v[pl.ds(16 * j, 16)] for j in range(_NV)]
        b_vec = [beta_v[pl.ds(16 * j, 16)] for j in range(_NV)]

        dnums = lax.GatherDimensionNumbers(
            offset_dims=(), collapsed_slice_dims=(0,), start_index_map=(0,))
        iota16 = lax.iota(jnp.int32, 16)
        perm_idx = [jnp.bitwise_xor(iota16, sh).reshape(16, 1)
                    for sh in (8, 4, 2, 1)]

        def _lane_bcast_sum(v):
            for pidx in perm_idx:
                v = v + lax.gather(
                    v, pidx, dnums, slice_sizes=(1,),
                    mode=lax.GatherScatterMode.PROMISE_IN_BOUNDS)
            return v

        def row_body(r, c2):
            h = [buf[r, pl.ds(16 * j, 16)] + bias_v[r, pl.ds(16 * j, 16)]
                 for j in range(_NV)]
            s = h[0]
            ss = h[0] * h[0]
            for j in range(1, _NV):
                s = s + h[j]
                ss = ss + h[j] * h[j]
            mean = _lane_bcast_sum(s) * (1.0 / _D)
            tot2 = _lane_bcast_sum(ss) * (1.0 / _D)
            x = tot2 - mean * mean + _EPS
            xi = lax.bitcast_convert_type(x, jnp.int32)
            yi = 0x5F3759DF - lax.shift_right_logical(xi, 1)
            y = lax.bitcast_convert_type(yi, jnp.float32)
            hx = x * 0.5
            y = y * (1.5 - hx * y * y)
            y = y * (1.5 - hx * y * y)
            y = y * (1.5 - hx * y * y)
            for j in range(_NV):
                buf[r, pl.ds(16 * j, 16)] = (h[j] - mean) * (y * g_vec[j]) + b_vec[j]
            return c2

        def chunk_body(g, c):
            off = g * T
            pltpu.async_copy(
                table_hbm.at[idx_v.at[pl.ds(off, T)]], buf, sem
            ).wait()
            lax.fori_loop(0, T, row_body, 0, unroll=2)
            pltpu.sync_copy(buf, out_hbm.at[pl.ds(base + off, T)])
            return c

        lax.fori_loop(0, n_chunks, chunk_body, 0)

    return fused_kernel


def kernel(input_ids, W_word, W_pos, W_tt, gamma, beta):
    B, T = input_ids.shape
    ids_flat = input_ids.reshape(-1).astype(jnp.int32)
    out = _make_fused(B * T, T)(ids_flat, W_word, W_pos, W_tt, gamma, beta)
    return out.reshape(B, T, _D)


# speedup vs baseline: 1.5800x; 1.5800x over previous
import functools

import jax
import jax.numpy as jnp
from jax import lax
from jax.experimental import pallas as pl
from jax.experimental.pallas import tpu as pltpu
from jax.experimental.pallas import tpu_sc as plsc

_NC = 2
_NS = 16
_NW = _NC * _NS
_D = 128
_NV = _D // 16
_EPS = 1e-12


def _make_fused(n_rows: int, T: int):
    rows_per_w = n_rows // _NW
    n_chunks = rows_per_w // T
    mesh = plsc.VectorSubcoreMesh(core_axis_name="c", subcore_axis_name="s")

    @functools.partial(
        pl.kernel,
        out_type=jax.ShapeDtypeStruct((n_rows, _D), jnp.float32),
        mesh=mesh,
        scratch_types=[
            pltpu.VMEM((rows_per_w,), jnp.int32),
            pltpu.VMEM((T, _D), jnp.float32),
            pltpu.VMEM((T, _D), jnp.float32),
            pltpu.VMEM((T, _D), jnp.float32),
            pltpu.VMEM((2, _D), jnp.float32),
            pltpu.VMEM((_D,), jnp.float32),
            pltpu.VMEM((_D,), jnp.float32),
            pltpu.SemaphoreType.DMA,
        ],
    )
    def fused_kernel(ids_hbm, table_hbm, pos_hbm, tt_hbm, gamma_hbm, beta_hbm,
                     out_hbm, idx_v, buf, obuf, bias_v, tt_v, gamma_v, beta_v,
                     sem):
        wid = lax.axis_index("s") * _NC + lax.axis_index("c")
        base = wid * rows_per_w
        pltpu.sync_copy(ids_hbm.at[pl.ds(base, rows_per_w)], idx_v)
        pltpu.sync_copy(pos_hbm.at[pl.ds(0, T)], bias_v)
        pltpu.sync_copy(tt_hbm, tt_v)
        pltpu.sync_copy(gamma_hbm, gamma_v)
        pltpu.sync_copy(beta_hbm, beta_v)

        tt_row = [tt_v[0, pl.ds(16 * j, 16)] for j in range(_NV)]

        def bias_body(r, c):
            for j in range(_NV):
                sl = pl.ds(16 * j, 16)
                bias_v[r, sl] = bias_v[r, sl] + tt_row[j]
            return c

        lax.fori_loop(0, T, bias_body, 0)

        g_vec = [gamma_v[pl.ds(16 * j, 16)] for j in range(_NV)]
        b_vec = [beta_v[pl.ds(16 * j, 16)] for j in range(_NV)]

        dnums = lax.GatherDimensionNumbers(
            offset_dims=(), collapsed_slice_dims=(0,), start_index_map=(0,))
        iota16 = lax.iota(jnp.int32, 16)
        perm_idx = [jnp.bitwise_xor(iota16, sh).reshape(16, 1)
                    for sh in (8, 4, 2, 1)]

        def _lane_bcast_sum(v):
            for pidx in perm_idx:
                v = v + lax.gather(
                    v, pidx, dnums, slice_sizes=(1,),
                    mode=lax.GatherScatterMode.PROMISE_IN_BOUNDS)
            return v

        def row_body(r):
            h = [buf[r, pl.ds(16 * j, 16)] + bias_v[r, pl.ds(16 * j, 16)]
                 for j in range(_NV)]
            s = h[0]
            ss = h[0] * h[0]
            for j in range(1, _NV):
                s = s + h[j]
                ss = ss + h[j] * h[j]
            mean = _lane_bcast_sum(s) * (1.0 / _D)
            tot2 = _lane_bcast_sum(ss) * (1.0 / _D)
            x = tot2 - mean * mean + _EPS
            xi = lax.bitcast_convert_type(x, jnp.int32)
            yi = 0x5F3759DF - lax.shift_right_logical(xi, 1)
            y = lax.bitcast_convert_type(yi, jnp.float32)
            hx = x * 0.5
            y = y * (1.5 - hx * y * y)
            y = y * (1.5 - hx * y * y)
            y = y * (1.5 - hx * y * y)
            for j in range(_NV):
                obuf[r, pl.ds(16 * j, 16)] = (h[j] - mean) * (y * g_vec[j]) + b_vec[j]

        def chunk_body(g, c):
            off = g * T
            pltpu.async_copy(
                table_hbm.at[idx_v.at[pl.ds(off, T)]], buf, sem
            ).wait()
            plsc.parallel_loop(0, T, 1, unroll=4)(row_body)
            pltpu.sync_copy(obuf, out_hbm.at[pl.ds(base + off, T)])
            return c

        lax.fori_loop(0, n_chunks, chunk_body, 0)

    return fused_kernel


def kernel(input_ids, W_word, W_pos, W_tt, gamma, beta):
    B, T = input_ids.shape
    ids_flat = input_ids.reshape(-1).astype(jnp.int32)
    out = _make_fused(B * T, T)(ids_flat, W_word, W_pos, W_tt, gamma, beta)
    return out.reshape(B, T, _D)
